# baseline (speedup 1.0000x reference)
import jax
import jax.numpy as jnp
from jax import lax
from jax.experimental import pallas as pl
from jax.experimental.pallas import tpu as pltpu
from jax.experimental.pallas import tpu_sc as plsc

SEQ = 128
DIM = 1280
NS = 8
RCH = SEQ // NS
IBLK = 8


def _sc_gather_body(table_hbm, idx_hbm, rows_hbm, idx_v, rows_v, sem):
    w = lax.axis_index("s")
    pltpu.sync_copy(idx_hbm.at[w], idx_v)
    pltpu.async_copy(table_hbm.at[idx_v], rows_v, sem).wait()
    pltpu.sync_copy(rows_v, rows_hbm.at[pl.ds(w * RCH, RCH)])


def _tc_broadcast_body(rows_ref, out_ref):
    out_ref[...] = jnp.broadcast_to(rows_ref[...], (IBLK, SEQ, DIM))


def kernel(inputs, length, table):
    del inputs
    idx = jnp.arange(SEQ, dtype=jnp.int32) + jnp.asarray(length, jnp.int32)
    idx = jnp.clip(idx, 0, SEQ - 1).reshape(NS, RCH)

    gather = pl.kernel(
        _sc_gather_body,
        mesh=plsc.VectorSubcoreMesh(
            core_axis_name="c", subcore_axis_name="s", num_cores=1, num_subcores=8
        ),
        out_type=jax.ShapeDtypeStruct((SEQ, DIM), jnp.float32),
        scratch_types=[
            pltpu.VMEM((RCH,), jnp.int32),
            pltpu.VMEM((RCH, DIM), jnp.float32),
            pltpu.SemaphoreType.DMA,
        ],
    )
    rows = gather(table, idx)

    return pl.pallas_call(
        _tc_broadcast_body,
        grid=(SEQ // IBLK,),
        in_specs=[pl.BlockSpec((SEQ, DIM), lambda i: (0, 0))],
        out_specs=pl.BlockSpec((IBLK, SEQ, DIM), lambda i: (i, 0, 0)),
        out_shape=jax.ShapeDtypeStruct((SEQ, SEQ, DIM), jnp.float32),
    )(rows)


# speedup vs baseline: 1.0220x; 1.0220x over previous
import jax
import jax.numpy as jnp
from jax import lax
from jax.experimental import pallas as pl
from jax.experimental.pallas import tpu as pltpu
from jax.experimental.pallas import tpu_sc as plsc

SEQ = 128
DIM = 1280
NS = 16
RCH = SEQ // NS
IBLK = 8


def _sc_gather_body(table_hbm, idx_hbm, rows_hbm, idx_v, rows_v, sem):
    w = lax.axis_index("s")
    pltpu.sync_copy(idx_hbm.at[w], idx_v)
    pltpu.async_copy(table_hbm.at[idx_v], rows_v, sem).wait()
    pltpu.sync_copy(rows_v, rows_hbm.at[pl.ds(w * RCH, RCH)])


def _tc_broadcast_body(rows_ref, out_ref):
    out_ref[...] = jnp.broadcast_to(rows_ref[...], (IBLK, SEQ, DIM))


def kernel(inputs, length, table):
    del inputs
    idx = jnp.arange(SEQ, dtype=jnp.int32) + jnp.asarray(length, jnp.int32)
    idx = jnp.clip(idx, 0, SEQ - 1).reshape(NS, RCH)

    gather = pl.kernel(
        _sc_gather_body,
        mesh=plsc.VectorSubcoreMesh(
            core_axis_name="c", subcore_axis_name="s", num_cores=1
        ),
        out_type=jax.ShapeDtypeStruct((SEQ, DIM), jnp.float32),
        compiler_params=pltpu.CompilerParams(skip_device_barrier=True),
        scratch_types=[
            pltpu.VMEM((RCH,), jnp.int32),
            pltpu.VMEM((RCH, DIM), jnp.float32),
            pltpu.SemaphoreType.DMA,
        ],
    )
    rows = gather(table, idx)

    return pl.pallas_call(
        _tc_broadcast_body,
        grid=(SEQ // IBLK,),
        in_specs=[pl.BlockSpec((SEQ, DIM), lambda i: (0, 0))],
        out_specs=pl.BlockSpec((IBLK, SEQ, DIM), lambda i: (i, 0, 0)),
        out_shape=jax.ShapeDtypeStruct((SEQ, SEQ, DIM), jnp.float32),
    )(rows)


# speedup vs baseline: 1.0234x; 1.0014x over previous
import jax
import jax.numpy as jnp
from jax import lax
from jax.experimental import pallas as pl
from jax.experimental.pallas import tpu as pltpu
from jax.experimental.pallas import tpu_sc as plsc

SEQ = 128
DIM = 1280
NS = 16
RCH = SEQ // NS
IBLK = 8


def _sc_gather_body(table_hbm, idx_hbm, rows_hbm, idx_v, rows_v, sem):
    w = lax.axis_index("s")
    pltpu.sync_copy(idx_hbm.at[w], idx_v)
    pltpu.async_copy(table_hbm.at[idx_v], rows_v, sem).wait()
    pltpu.sync_copy(rows_v, rows_hbm.at[pl.ds(w * RCH, RCH)])


def _tc_broadcast_body(rows_ref, out_ref):
    out_ref[...] = jnp.broadcast_to(rows_ref[...], (IBLK, SEQ, DIM))


def kernel(inputs, length, table):
    del inputs
    idx = jnp.arange(SEQ, dtype=jnp.int32) + jnp.asarray(length, jnp.int32)
    idx = jnp.clip(idx, 0, SEQ - 1).reshape(NS, RCH)

    gather = pl.kernel(
        _sc_gather_body,
        mesh=plsc.VectorSubcoreMesh(
            core_axis_name="c", subcore_axis_name="s", num_cores=1
        ),
        out_type=jax.ShapeDtypeStruct((SEQ, DIM), jnp.float32),
        scratch_types=[
            pltpu.VMEM((RCH,), jnp.int32),
            pltpu.VMEM((RCH, DIM), jnp.float32),
            pltpu.SemaphoreType.DMA,
        ],
    )
    rows = gather(table, idx)

    return pl.pallas_call(
        _tc_broadcast_body,
        grid=(SEQ // IBLK,),
        in_specs=[pl.BlockSpec((SEQ, DIM), lambda i: (0, 0))],
        out_specs=pl.BlockSpec((IBLK, SEQ, DIM), lambda i: (i, 0, 0)),
        out_shape=jax.ShapeDtypeStruct((SEQ, SEQ, DIM), jnp.float32),
    )(rows)
